# Initial kernel scaffold; baseline (speedup 1.0000x reference)
import jax
import jax.numpy as jnp
from jax.experimental import pallas as pl


def kernel(input_ids, embed_table):
    raise NotImplementedError("write your pallas kernel here")


# speedup vs baseline: 5.2565x; 5.2565x over previous
import functools

import jax
import jax.numpy as jnp
from jax import lax
from jax.experimental import pallas as pl
from jax.experimental.pallas import tpu as pltpu
from jax.experimental.pallas import tpu_sc as plsc

_ROWS, _D = 8, 4
_N = 16384 * 200
_NC, _NS, _L = 2, 16, 16
_NW = _NC * _NS
_PER_W = _N // _NW
_CHUNK = 10240
_STEPS = _CHUNK // _L
_ITERS = _PER_W // _CHUNK


def _body(ids_hbm, tab_hbm, out_hbm,
          tab_v, ids_v0, ids_v1, out_v0, out_v1, sin0, sin1, sout0, sout1):
    wid = lax.axis_index("s") * _NC + lax.axis_index("c")
    base = wid * _PER_W

    pltpu.sync_copy(tab_hbm, tab_v)

    iota = lax.iota(jnp.int32, _L)
    lane4 = iota * _D

    ids_bufs = (ids_v0, ids_v1)
    out_bufs = (out_v0, out_v1)
    in_sems = (sin0, sin1)
    out_sems = (sout0, sout1)

    def compute(b):
        ids_b = ids_bufs[b]
        out_b = out_bufs[b]

        def step(i, carry):
            off = pl.multiple_of(i * _L, _L)
            idvec = ids_b[pl.ds(off, _L)]
            tidx = idvec * _D
            pos = lane4 + i * (_L * _D)
            for c in range(_D):
                val = plsc.load_gather(tab_v, [tidx + c])
                plsc.store_scatter(out_b, [pos + c], val)
            return carry

        lax.fori_loop(0, _STEPS, step, 0)

    def in_copy(g, b):
        return pltpu.async_copy(
            ids_hbm.at[pl.ds(base + g * _CHUNK, _CHUNK)], ids_bufs[b], in_sems[b])

    def out_copy(g, b):
        return pltpu.async_copy(
            out_bufs[b], out_hbm.at[pl.ds((base + g * _CHUNK) * _D, _CHUNK * _D)],
            out_sems[b])

    pending_in = {0: in_copy(0, 0)}
    pending_out = {}
    for g in range(_ITERS):
        b = g & 1
        if g + 1 < _ITERS:
            pending_in[(g + 1) & 1] = in_copy(g + 1, (g + 1) & 1)
        pending_in[b].wait()
        if b in pending_out:
            pending_out[b].wait()
        compute(b)
        pending_out[b] = out_copy(g, b)
    for b in (0, 1):
        if b in pending_out:
            pending_out[b].wait()


@functools.partial(
    pl.kernel,
    out_type=jax.ShapeDtypeStruct((_N * _D,), jnp.float32),
    mesh=plsc.VectorSubcoreMesh(core_axis_name="c", subcore_axis_name="s"),
    compiler_params=pltpu.CompilerParams(needs_layout_passes=False),
    scratch_types=[
        pltpu.VMEM((_ROWS * _D,), jnp.float32),
        pltpu.VMEM((_CHUNK,), jnp.int32),
        pltpu.VMEM((_CHUNK,), jnp.int32),
        pltpu.VMEM((_CHUNK * _D,), jnp.float32),
        pltpu.VMEM((_CHUNK * _D,), jnp.float32),
        pltpu.SemaphoreType.DMA,
        pltpu.SemaphoreType.DMA,
        pltpu.SemaphoreType.DMA,
        pltpu.SemaphoreType.DMA,
    ],
)
def _embed_sc(ids_hbm, tab_hbm, out_hbm, *scratch):
    _body(ids_hbm, tab_hbm, out_hbm, *scratch)


def kernel(input_ids, embed_table):
    ids_flat = input_ids.reshape(-1).astype(jnp.int32)
    tab_flat = embed_table.reshape(-1)
    out = _embed_sc(ids_flat, tab_flat)
    return out.reshape(input_ids.shape + (_D,))


# speedup vs baseline: 5.5124x; 1.0487x over previous
import functools

import jax
import jax.numpy as jnp
from jax import lax
from jax.experimental import pallas as pl
from jax.experimental.pallas import tpu as pltpu
from jax.experimental.pallas import tpu_sc as plsc

_ROWS, _D = 8, 4
_N = 16384 * 200
_NC, _NS, _L = 2, 16, 16
_NW = _NC * _NS
_PER_W = _N // _NW
_CHUNK = 10240
_STEPS = _CHUNK // _L
_ITERS = _PER_W // _CHUNK


def _body(ids_hbm, tab_hbm, out_hbm,
          tab_v, ids_v0, ids_v1, out_v0, out_v1, sin0, sin1, sout0, sout1):
    wid = lax.axis_index("s") * _NC + lax.axis_index("c")
    base = wid * _PER_W

    pltpu.sync_copy(tab_hbm, tab_v)

    iota = lax.iota(jnp.int32, _L)
    lane4 = iota * _D

    ids_bufs = (ids_v0, ids_v1)
    out_bufs = (out_v0, out_v1)
    in_sems = (sin0, sin1)
    out_sems = (sout0, sout1)

    def compute(b):
        ids_b = ids_bufs[b]
        out_b = out_bufs[b]

        @plsc.parallel_loop(0, _STEPS, unroll=8)
        def step(i):
            off = pl.multiple_of(i * _L, _L)
            idvec = ids_b[pl.ds(off, _L)]
            tidx = idvec * _D
            pos = lane4 + i * (_L * _D)
            for c in range(_D):
                val = plsc.load_gather(tab_v, [tidx + c])
                plsc.store_scatter(out_b, [pos + c], val)

    def in_copy(g, b):
        return pltpu.async_copy(
            ids_hbm.at[pl.ds(base + g * _CHUNK, _CHUNK)], ids_bufs[b], in_sems[b])

    def out_copy(g, b):
        return pltpu.async_copy(
            out_bufs[b], out_hbm.at[pl.ds((base + g * _CHUNK) * _D, _CHUNK * _D)],
            out_sems[b])

    pending_in = {0: in_copy(0, 0)}
    pending_out = {}
    for g in range(_ITERS):
        b = g & 1
        if g + 1 < _ITERS:
            pending_in[(g + 1) & 1] = in_copy(g + 1, (g + 1) & 1)
        pending_in[b].wait()
        if b in pending_out:
            pending_out[b].wait()
        compute(b)
        pending_out[b] = out_copy(g, b)
    for b in (0, 1):
        if b in pending_out:
            pending_out[b].wait()


@functools.partial(
    pl.kernel,
    out_type=jax.ShapeDtypeStruct((_N * _D,), jnp.float32),
    mesh=plsc.VectorSubcoreMesh(core_axis_name="c", subcore_axis_name="s"),
    compiler_params=pltpu.CompilerParams(needs_layout_passes=False),
    scratch_types=[
        pltpu.VMEM((_ROWS * _D,), jnp.float32),
        pltpu.VMEM((_CHUNK,), jnp.int32),
        pltpu.VMEM((_CHUNK,), jnp.int32),
        pltpu.VMEM((_CHUNK * _D,), jnp.float32),
        pltpu.VMEM((_CHUNK * _D,), jnp.float32),
        pltpu.SemaphoreType.DMA,
        pltpu.SemaphoreType.DMA,
        pltpu.SemaphoreType.DMA,
        pltpu.SemaphoreType.DMA,
    ],
)
def _embed_sc(ids_hbm, tab_hbm, out_hbm, *scratch):
    _body(ids_hbm, tab_hbm, out_hbm, *scratch)


def kernel(input_ids, embed_table):
    ids_flat = input_ids.reshape(-1).astype(jnp.int32)
    tab_flat = embed_table.reshape(-1)
    out = _embed_sc(ids_flat, tab_flat)
    return out.reshape(input_ids.shape + (_D,))


# speedup vs baseline: 228.9040x; 41.5250x over previous
import functools

import jax
import jax.numpy as jnp
from jax import lax
from jax.experimental import pallas as pl
from jax.experimental.pallas import tpu as pltpu
from jax.experimental.pallas import tpu_sc as plsc

_ROWS, _D = 8, 4
_NI, _NJ = 16384, 200
_N = _NI * _NJ
_NC, _NS, _L = 2, 16, 16
_NW = _NC * _NS
_NJ1 = _NJ // 8
_NI1 = _NI // 128
_I1W = _NI1 // _NW
_CH_IN = _I1W * 1024
_CH_OUT = _CH_IN * _D
_STEPS = _CH_IN // _L


def _body(ids_hbm, tab_hbm, out_hbm,
          tab_v, ids_v0, ids_v1, out_v0, out_v1, sin0, sin1, sout0, sout1):
    wid = lax.axis_index("s") * _NC + lax.axis_index("c")

    pltpu.sync_copy(tab_hbm, tab_v)

    ids_bufs = (ids_v0, ids_v1)
    out_bufs = (out_v0, out_v1)
    in_sems = (sin0, sin1)
    out_sems = (sout0, sout1)

    def compute(b):
        ids_b = ids_bufs[b]
        out_b = out_bufs[b]

        @plsc.parallel_loop(0, _STEPS, unroll=8)
        def step(s):
            i1r = s >> 6
            j0 = (s >> 3) & 7
            i0b = s & 7
            in_off = (i1r << 10) | (j0 << 7) | (i0b << 4)
            out_off = (j0 << 11) | (i1r << 9) | (i0b << 4)
            idvec = ids_b[pl.ds(pl.multiple_of(in_off, _L), _L)]
            idx4 = idvec * _D
            for d in range(_D):
                val = plsc.load_gather(tab_v, [idx4 + d])
                out_b[pl.ds(pl.multiple_of(out_off + (d << 7), _L), _L)] = val

    def in_copy(g, b):
        return pltpu.async_copy(
            ids_hbm.at[pl.ds(g * (_NI1 * 1024) + wid * _CH_IN, _CH_IN)],
            ids_bufs[b], in_sems[b])

    def out_copy(g, b):
        cps = []
        for j0 in range(8):
            dst = out_hbm.at[pl.ds((g * 8 + j0) * (_NI1 * 512) + wid * 2048, 2048)]
            cps.append(pltpu.async_copy(
                out_bufs[b].at[pl.ds(j0 * 2048, 2048)], dst, out_sems[b]))
        return cps

    pending_in = {0: in_copy(0, 0)}
    pending_out = {}
    for g in range(_NJ1):
        b = g & 1
        if g + 1 < _NJ1:
            pending_in[(g + 1) & 1] = in_copy(g + 1, (g + 1) & 1)
        pending_in[b].wait()
        for cp in pending_out.pop(b, ()):
            cp.wait()
        compute(b)
        pending_out[b] = out_copy(g, b)
    for b in (0, 1):
        for cp in pending_out.pop(b, ()):
            cp.wait()


@functools.partial(
    pl.kernel,
    out_type=jax.ShapeDtypeStruct((_N * _D,), jnp.float32),
    mesh=plsc.VectorSubcoreMesh(core_axis_name="c", subcore_axis_name="s"),
    compiler_params=pltpu.CompilerParams(needs_layout_passes=False),
    scratch_types=[
        pltpu.VMEM((_ROWS * _D,), jnp.float32),
        pltpu.VMEM((_CH_IN,), jnp.int32),
        pltpu.VMEM((_CH_IN,), jnp.int32),
        pltpu.VMEM((_CH_OUT,), jnp.float32),
        pltpu.VMEM((_CH_OUT,), jnp.float32),
        pltpu.SemaphoreType.DMA,
        pltpu.SemaphoreType.DMA,
        pltpu.SemaphoreType.DMA,
        pltpu.SemaphoreType.DMA,
    ],
)
def _embed_sc(ids_hbm, tab_hbm, out_hbm, *scratch):
    _body(ids_hbm, tab_hbm, out_hbm, *scratch)


def kernel(input_ids, embed_table):
    ids = input_ids.astype(jnp.int32)
    ids_phys = (ids.T.reshape(_NJ1, 8, _NI1, 128)
                .transpose(0, 2, 1, 3).reshape(-1))
    tab_flat = embed_table.reshape(-1)
    flat = _embed_sc(ids_phys, tab_flat)
    out = (flat.reshape(_NJ, _NI1, _D, 128)
           .transpose(1, 3, 0, 2).reshape(_NI, _NJ, _D))
    return out
